# baseline (speedup 1.0000x reference)
import jax
import jax.numpy as jnp
from jax.experimental import pallas as pl


def _body(x_ref, o_ref):
    for rh in range(2):
        for rw in range(2):
            v = x_ref[0, 0, pl.ds(rh, 32, 2), pl.ds(rw, 32, 2), :]
            for rz in range(2):
                idx = jax.lax.broadcasted_iota(jnp.int32, (32, 32, 32), 2) * 2 + rz
                o_ref[0, (rh * 2 + rw) * 2 + rz] = jnp.take_along_axis(
                    v, idx, axis=-1
                )


def kernel(x):
    B, C, H, W, Z = x.shape
    r = 2
    return pl.pallas_call(
        _body,
        grid=(B, C),
        in_specs=[pl.BlockSpec((1, 1, H, W, Z), lambda b, c: (b, c, 0, 0, 0))],
        out_specs=pl.BlockSpec(
            (1, r**3, H // r, W // r, Z // r),
            lambda b, c: (b, c, 0, 0, 0),
        ),
        out_shape=jax.ShapeDtypeStruct(
            (B, C * r**3, H // r, W // r, Z // r), x.dtype
        ),
    )(x)


# speedup vs baseline: 1.1475x; 1.1475x over previous
import jax
import jax.numpy as jnp
from jax.experimental import pallas as pl


def _body(x_ref, o_ref):
    d = jax.lax.broadcasted_iota(jnp.int32, (32, 32, 128), 2)
    src = (d // 64) * 64 + 2 * (d % 32) + (d % 64) // 32
    for rh in range(2):
        vh = x_ref[0, 0, pl.ds(rh, 32, 2), :, :]
        g1 = jnp.take_along_axis(vh, src, axis=-1)
        for p in range(4):
            t = g1[:, :, 32 * p : 32 * p + 32]
            o_ref[0, 4 * rh + p] = t.reshape(32, 1024)


def kernel(x):
    B, C, H, W, Z = x.shape
    r = 2
    x2 = x.reshape(B, C, H, W // r, r * Z)
    out = pl.pallas_call(
        _body,
        grid=(B, C),
        in_specs=[
            pl.BlockSpec((1, 1, H, W // r, r * Z), lambda b, c: (b, c, 0, 0, 0))
        ],
        out_specs=pl.BlockSpec(
            (1, r**3, H // r, (W // r) * (Z // r)),
            lambda b, c: (b, c, 0, 0),
        ),
        out_shape=jax.ShapeDtypeStruct(
            (B, C * r**3, H // r, (W // r) * (Z // r)), x.dtype
        ),
    )(x2)
    return out.reshape(B, C * r**3, H // r, W // r, Z // r)
